# baseline (speedup 1.0000x reference)
import functools
import math

import jax
import jax.numpy as jnp
from jax import lax
from jax.experimental import pallas as pl
from jax.experimental.pallas import tpu as pltpu, tpu_sc as plsc

D_MODEL = 64
SCALE = math.sqrt(D_MODEL)

_info = plsc.get_sparse_core_info()
NC, NS, L = _info.num_cores, _info.num_subcores, _info.num_lanes
NW = NC * NS


def _make_kernel(B, D, chunk):
    assert B % (NW * chunk) == 0
    b_per_w = B // NW
    steps = b_per_w // chunk
    mesh = plsc.VectorSubcoreMesh(core_axis_name="c", subcore_axis_name="s")

    @functools.partial(
        pl.kernel,
        mesh=mesh,
        out_type=jax.ShapeDtypeStruct((B, D), jnp.float32),
        scratch_types=[
            pltpu.VMEM((chunk,), jnp.int32),
            pltpu.VMEM((chunk, D), jnp.float32),
            pltpu.SemaphoreType.DMA,
        ],
        compiler_params=pltpu.CompilerParams(use_tc_tiling_on_sc=False),
    )
    def k(x_hbm, table_hbm, out_hbm, idx_v, rows_v, sem):
        wid = lax.axis_index("s") * NC + lax.axis_index("c")
        wbase = wid * b_per_w

        def step(s, carry):
            base = wbase + s * chunk
            pltpu.sync_copy(x_hbm.at[pl.ds(base, chunk)], idx_v)
            pltpu.async_copy(table_hbm.at[idx_v], rows_v, sem).wait()

            def scale_row(r, c):
                for j in range(D // L):
                    sl = pl.ds(j * L, L)
                    rows_v[r, sl] = rows_v[r, sl] * SCALE
                return c

            lax.fori_loop(0, chunk, scale_row, 0)
            pltpu.sync_copy(rows_v, out_hbm.at[pl.ds(base, chunk)])
            return carry

        lax.fori_loop(0, steps, step, 0)

    return k


@jax.jit
def kernel(x, table):
    batch, seq = x.shape
    B = batch * seq
    idx = x.reshape(B).astype(jnp.int32)
    out = _make_kernel(B, D_MODEL, 512)(idx, table)
    return out.reshape(batch, seq, D_MODEL)


# speedup vs baseline: 1.1324x; 1.1324x over previous
import functools
import math

import jax
import jax.numpy as jnp
from jax import lax
from jax.experimental import pallas as pl
from jax.experimental.pallas import tpu as pltpu, tpu_sc as plsc

D_MODEL = 64
SCALE = math.sqrt(D_MODEL)

_info = plsc.get_sparse_core_info()
NC, NS, L = _info.num_cores, _info.num_subcores, _info.num_lanes
NW = NC * NS


def _make_kernel(B, D, chunk):
    assert B % (NW * chunk) == 0
    b_per_w = B // NW
    steps = b_per_w // chunk
    assert steps % 2 == 0 and steps >= 4
    unroll_rows = 8
    assert chunk % unroll_rows == 0
    mesh = plsc.VectorSubcoreMesh(core_axis_name="c", subcore_axis_name="s")

    @functools.partial(
        pl.kernel,
        mesh=mesh,
        out_type=jax.ShapeDtypeStruct((B, D), jnp.float32),
        scratch_types=[
            pltpu.VMEM((chunk,), jnp.int32),
            pltpu.VMEM((chunk,), jnp.int32),
            pltpu.VMEM((chunk, D), jnp.float32),
            pltpu.VMEM((chunk, D), jnp.float32),
            pltpu.SemaphoreType.DMA,
            pltpu.SemaphoreType.DMA,
            pltpu.SemaphoreType.DMA,
            pltpu.SemaphoreType.DMA,
        ],
        compiler_params=pltpu.CompilerParams(use_tc_tiling_on_sc=False),
    )
    def k(x_hbm, table_hbm, out_hbm, idx0, idx1, rows0, rows1, g0, g1, w0,
          w1):
        wid = lax.axis_index("s") * NC + lax.axis_index("c")
        wbase = wid * b_per_w
        idx = (idx0, idx1)
        rows = (rows0, rows1)
        gsem = (g0, g1)
        wsem = (w0, w1)

        def start_gather(s, p):
            pltpu.sync_copy(x_hbm.at[pl.ds(wbase + s * chunk, chunk)],
                            idx[p])
            pltpu.async_copy(table_hbm.at[idx[p]], rows[p], gsem[p])

        def scale(p):
            buf = rows[p]

            def body(i, c):
                r0 = i * unroll_rows
                for r in range(unroll_rows):
                    for j in range(D // L):
                        sl = pl.ds(j * L, L)
                        buf[r0 + r, sl] = buf[r0 + r, sl] * SCALE
                return c

            lax.fori_loop(0, chunk // unroll_rows, body, 0)

        def wait_gather(p):
            pltpu.make_async_copy(table_hbm.at[idx[p]], rows[p],
                                  gsem[p]).wait()

        def start_write(s, p):
            pltpu.async_copy(rows[p], out_hbm.at[pl.ds(wbase + s * chunk,
                                                       chunk)], wsem[p])

        def wait_write(s, p):
            pltpu.make_async_copy(rows[p],
                                  out_hbm.at[pl.ds(wbase + s * chunk, chunk)],
                                  wsem[p]).wait()

        start_gather(0, 0)
        wait_gather(0)
        start_gather(1, 1)
        scale(0)
        start_write(0, 0)

        def pair(jj, c):
            s = 1 + 2 * jj
            for p in (1, 0):
                q = 1 - p
                wait_gather(p)
                wait_write(s - 1, q)
                start_gather(s + 1, q)
                scale(p)
                start_write(s, p)
                s += 1
            return c

        lax.fori_loop(0, (steps - 2) // 2, pair, 0)

        wait_gather(1)
        scale(1)
        start_write(steps - 1, 1)
        wait_write(steps - 2, 0)
        wait_write(steps - 1, 1)

    return k


@jax.jit
def kernel(x, table):
    batch, seq = x.shape
    B = batch * seq
    idx = x.reshape(B).astype(jnp.int32)
    out = _make_kernel(B, D_MODEL, 800)(idx, table)
    return out.reshape(batch, seq, D_MODEL)


# speedup vs baseline: 1.2146x; 1.0726x over previous
import functools
import math

import jax
import jax.numpy as jnp
from jax import lax
from jax.experimental import pallas as pl
from jax.experimental.pallas import tpu as pltpu, tpu_sc as plsc

D_MODEL = 64
DPAD = 128
SCALE = math.sqrt(D_MODEL)

_info = plsc.get_sparse_core_info()
NC, NS, L = _info.num_cores, _info.num_subcores, _info.num_lanes
NW = NC * NS


def _make_kernel(B, chunk):
    assert B % (NW * chunk) == 0
    b_per_w = B // NW
    steps = b_per_w // chunk
    assert steps % 2 == 0 and steps >= 4
    unroll_rows = 8
    assert chunk % unroll_rows == 0
    mesh = plsc.VectorSubcoreMesh(core_axis_name="c", subcore_axis_name="s")

    @functools.partial(
        pl.kernel,
        mesh=mesh,
        out_type=jax.ShapeDtypeStruct((B, D_MODEL), jnp.float32),
        scratch_types=[
            pltpu.VMEM((chunk,), jnp.int32),
            pltpu.VMEM((chunk,), jnp.int32),
            pltpu.VMEM((chunk, DPAD), jnp.float32),
            pltpu.VMEM((chunk, DPAD), jnp.float32),
            pltpu.VMEM((chunk, D_MODEL), jnp.float32),
            pltpu.VMEM((chunk, D_MODEL), jnp.float32),
            pltpu.SemaphoreType.DMA,
            pltpu.SemaphoreType.DMA,
            pltpu.SemaphoreType.DMA,
            pltpu.SemaphoreType.DMA,
        ],
        compiler_params=pltpu.CompilerParams(use_tc_tiling_on_sc=True),
    )
    def k(x_hbm, table_hbm, out_hbm, idx0, idx1, rows0, rows1, cmp0, cmp1,
          g0, g1, w0, w1):
        wid = lax.axis_index("s") * NC + lax.axis_index("c")
        wbase = wid * b_per_w
        idx = (idx0, idx1)
        rows = (rows0, rows1)
        cmp = (cmp0, cmp1)
        gsem = (g0, g1)
        wsem = (w0, w1)

        def start_gather(s, p):
            pltpu.sync_copy(x_hbm.at[pl.ds(wbase + s * chunk, chunk)],
                            idx[p])
            pltpu.async_copy(table_hbm.at[idx[p]], rows[p], gsem[p])

        def scale(p):
            src, dst = rows[p], cmp[p]

            def body(i, c):
                r0 = i * unroll_rows
                for r in range(unroll_rows):
                    for j in range(D_MODEL // L):
                        sl = pl.ds(j * L, L)
                        dst[r0 + r, sl] = src[r0 + r, sl] * SCALE
                return c

            lax.fori_loop(0, chunk // unroll_rows, body, 0)

        def wait_gather(p):
            pltpu.make_async_copy(table_hbm.at[idx[p]], rows[p],
                                  gsem[p]).wait()

        def out_slice(s):
            return out_hbm.at[pl.ds(wbase + s * chunk, chunk)]

        def src_slice(p):
            return cmp[p]

        def start_write(s, p):
            pltpu.async_copy(src_slice(p), out_slice(s), wsem[p])

        def wait_write(s, p):
            pltpu.make_async_copy(src_slice(p), out_slice(s), wsem[p]).wait()

        start_gather(0, 0)
        wait_gather(0)
        start_gather(1, 1)
        scale(0)
        start_write(0, 0)

        def pair(jj, c):
            s = 1 + 2 * jj
            for p in (1, 0):
                q = 1 - p
                wait_gather(p)
                wait_write(s - 1, q)
                start_gather(s + 1, q)
                scale(p)
                start_write(s, p)
                s += 1
            return c

        lax.fori_loop(0, (steps - 2) // 2, pair, 0)

        wait_gather(1)
        scale(1)
        start_write(steps - 1, 1)
        wait_write(steps - 2, 0)
        wait_write(steps - 1, 1)

    return k


@jax.jit
def kernel(x, table):
    batch, seq = x.shape
    B = batch * seq
    idx = x.reshape(B).astype(jnp.int32)
    table_p = jnp.pad(table, ((0, 0), (0, DPAD - D_MODEL)))
    out = _make_kernel(B, 128)(idx, table_p)
    return out.reshape(batch, seq, D_MODEL)


# speedup vs baseline: 1.3763x; 1.1331x over previous
import functools
import math

import jax
import jax.numpy as jnp
from jax import lax
from jax.experimental import pallas as pl
from jax.experimental.pallas import tpu as pltpu, tpu_sc as plsc

D_MODEL = 64
DPAD = 128
SCALE = math.sqrt(D_MODEL)
VBLOCK = 2048

_info = plsc.get_sparse_core_info()
NC, NS, L = _info.num_cores, _info.num_subcores, _info.num_lanes
NW = NC * NS


def _repack_body(tT_ref, out_ref):
    blk = tT_ref[...].T * SCALE
    out_ref[...] = jnp.concatenate(
        [blk, jnp.zeros((VBLOCK, DPAD - D_MODEL), jnp.float32)], axis=1)


def _repack(tT):
    V = tT.shape[1]
    grid = pl.cdiv(V, VBLOCK)
    return pl.pallas_call(
        _repack_body,
        grid=(grid,),
        in_specs=[pl.BlockSpec((D_MODEL, VBLOCK), lambda i: (0, i))],
        out_specs=pl.BlockSpec((VBLOCK, DPAD), lambda i: (i, 0)),
        out_shape=jax.ShapeDtypeStruct((V, DPAD), jnp.float32),
    )(tT)


def _make_gather(B, chunk):
    assert B % (NW * chunk) == 0
    b_per_w = B // NW
    steps = b_per_w // chunk
    assert steps % 2 == 0 and steps >= 4
    unroll_rows = 8
    assert chunk % unroll_rows == 0
    mesh = plsc.VectorSubcoreMesh(core_axis_name="c", subcore_axis_name="s")

    @functools.partial(
        pl.kernel,
        mesh=mesh,
        out_type=jax.ShapeDtypeStruct((B, D_MODEL), jnp.float32),
        scratch_types=[
            pltpu.VMEM((chunk,), jnp.int32),
            pltpu.VMEM((chunk,), jnp.int32),
            pltpu.VMEM((chunk, DPAD), jnp.float32),
            pltpu.VMEM((chunk, DPAD), jnp.float32),
            pltpu.VMEM((chunk, D_MODEL), jnp.float32),
            pltpu.VMEM((chunk, D_MODEL), jnp.float32),
            pltpu.SemaphoreType.DMA,
            pltpu.SemaphoreType.DMA,
            pltpu.SemaphoreType.DMA,
            pltpu.SemaphoreType.DMA,
        ],
        compiler_params=pltpu.CompilerParams(use_tc_tiling_on_sc=True),
    )
    def k(x_hbm, table_hbm, out_hbm, idx0, idx1, rows0, rows1, cmp0, cmp1,
          g0, g1, w0, w1):
        wid = lax.axis_index("s") * NC + lax.axis_index("c")
        wbase = wid * b_per_w
        idx = (idx0, idx1)
        rows = (rows0, rows1)
        cmp = (cmp0, cmp1)
        gsem = (g0, g1)
        wsem = (w0, w1)

        def start_gather(s, p):
            pltpu.sync_copy(x_hbm.at[pl.ds(wbase + s * chunk, chunk)],
                            idx[p])
            pltpu.async_copy(table_hbm.at[idx[p]], rows[p], gsem[p])

        def compact(p):
            src, dst = rows[p], cmp[p]

            def body(i, c):
                r0 = i * unroll_rows
                for r in range(unroll_rows):
                    for j in range(D_MODEL // L):
                        sl = pl.ds(j * L, L)
                        dst[r0 + r, sl] = src[r0 + r, sl]
                return c

            lax.fori_loop(0, chunk // unroll_rows, body, 0)

        def wait_gather(p):
            pltpu.make_async_copy(table_hbm.at[idx[p]], rows[p],
                                  gsem[p]).wait()

        def out_slice(s):
            return out_hbm.at[pl.ds(wbase + s * chunk, chunk)]

        def start_write(s, p):
            pltpu.async_copy(cmp[p], out_slice(s), wsem[p])

        def wait_write(s, p):
            pltpu.make_async_copy(cmp[p], out_slice(s), wsem[p]).wait()

        start_gather(0, 0)
        wait_gather(0)
        start_gather(1, 1)
        compact(0)
        start_write(0, 0)

        def pair(jj, c):
            s = 1 + 2 * jj
            for p in (1, 0):
                q = 1 - p
                wait_gather(p)
                wait_write(s - 1, q)
                start_gather(s + 1, q)
                compact(p)
                start_write(s, p)
                s += 1
            return c

        lax.fori_loop(0, (steps - 2) // 2, pair, 0)

        wait_gather(1)
        compact(1)
        start_write(steps - 1, 1)
        wait_write(steps - 2, 0)
        wait_write(steps - 1, 1)

    return k


@jax.jit
def kernel(x, table):
    batch, seq = x.shape
    B = batch * seq
    idx = x.reshape(B).astype(jnp.int32)
    table_p = _repack(table.T)
    out = _make_gather(B, 200)(idx, table_p)
    return out.reshape(batch, seq, D_MODEL)


# speedup vs baseline: 1.3809x; 1.0033x over previous
import functools
import math

import jax
import jax.numpy as jnp
from jax import lax
from jax.experimental import pallas as pl
from jax.experimental.pallas import tpu as pltpu, tpu_sc as plsc

D_MODEL = 64
DPAD = 128
SCALE = math.sqrt(D_MODEL)
VBLOCK = 2048

_info = plsc.get_sparse_core_info()
NC, NS, L = _info.num_cores, _info.num_subcores, _info.num_lanes
NW = NC * NS


def _repack_body(tT_ref, out_ref):
    out_ref[:, 0:D_MODEL] = tT_ref[...].T * SCALE


def _repack(tT):
    V = tT.shape[1]
    grid = pl.cdiv(V, VBLOCK)
    return pl.pallas_call(
        _repack_body,
        grid=(grid,),
        in_specs=[pl.BlockSpec((D_MODEL, VBLOCK), lambda i: (0, i))],
        out_specs=pl.BlockSpec((VBLOCK, DPAD), lambda i: (i, 0)),
        out_shape=jax.ShapeDtypeStruct((V, DPAD), jnp.float32),
    )(tT)


def _make_gather(B, chunk):
    assert B % (NW * chunk) == 0
    b_per_w = B // NW
    steps = b_per_w // chunk
    assert steps % 2 == 0 and steps >= 4
    unroll_rows = 8
    assert chunk % unroll_rows == 0
    mesh = plsc.VectorSubcoreMesh(core_axis_name="c", subcore_axis_name="s")

    @functools.partial(
        pl.kernel,
        mesh=mesh,
        out_type=jax.ShapeDtypeStruct((B, D_MODEL), jnp.float32),
        scratch_types=[
            pltpu.VMEM((chunk,), jnp.int32),
            pltpu.VMEM((chunk,), jnp.int32),
            pltpu.VMEM((chunk, DPAD), jnp.float32),
            pltpu.VMEM((chunk, DPAD), jnp.float32),
            pltpu.VMEM((chunk, D_MODEL), jnp.float32),
            pltpu.VMEM((chunk, D_MODEL), jnp.float32),
            pltpu.SemaphoreType.DMA,
            pltpu.SemaphoreType.DMA,
            pltpu.SemaphoreType.DMA,
            pltpu.SemaphoreType.DMA,
        ],
        compiler_params=pltpu.CompilerParams(use_tc_tiling_on_sc=True),
    )
    def k(x_hbm, table_hbm, out_hbm, idx0, idx1, rows0, rows1, cmp0, cmp1,
          g0, g1, w0, w1):
        wid = lax.axis_index("s") * NC + lax.axis_index("c")
        wbase = wid * b_per_w
        idx = (idx0, idx1)
        rows = (rows0, rows1)
        cmp = (cmp0, cmp1)
        gsem = (g0, g1)
        wsem = (w0, w1)

        def start_gather(s, p):
            pltpu.sync_copy(x_hbm.at[pl.ds(wbase + s * chunk, chunk)],
                            idx[p])
            pltpu.async_copy(table_hbm.at[idx[p]], rows[p], gsem[p])

        def compact(p):
            src, dst = rows[p], cmp[p]

            def body(i, c):
                r0 = i * unroll_rows
                for r in range(unroll_rows):
                    for j in range(D_MODEL // L):
                        sl = pl.ds(j * L, L)
                        dst[r0 + r, sl] = src[r0 + r, sl]
                return c

            lax.fori_loop(0, chunk // unroll_rows, body, 0)

        def wait_gather(p):
            pltpu.make_async_copy(table_hbm.at[idx[p]], rows[p],
                                  gsem[p]).wait()

        def out_slice(s):
            return out_hbm.at[pl.ds(wbase + s * chunk, chunk)]

        def start_write(s, p):
            pltpu.async_copy(cmp[p], out_slice(s), wsem[p])

        def wait_write(s, p):
            pltpu.make_async_copy(cmp[p], out_slice(s), wsem[p]).wait()

        start_gather(0, 0)
        wait_gather(0)
        start_gather(1, 1)
        compact(0)
        start_write(0, 0)

        def pair(jj, c):
            s = 1 + 2 * jj
            for p in (1, 0):
                q = 1 - p
                wait_gather(p)
                wait_write(s - 1, q)
                start_gather(s + 1, q)
                compact(p)
                start_write(s, p)
                s += 1
            return c

        lax.fori_loop(0, (steps - 2) // 2, pair, 0)

        wait_gather(1)
        compact(1)
        start_write(steps - 1, 1)
        wait_write(steps - 2, 0)
        wait_write(steps - 1, 1)

    return k


@jax.jit
def kernel(x, table):
    batch, seq = x.shape
    B = batch * seq
    idx = x.reshape(B).astype(jnp.int32)
    table_p = _repack(table.T)
    out = _make_gather(B, 200)(idx, table_p)
    return out.reshape(batch, seq, D_MODEL)
